# baseline (speedup 1.0000x reference)
import functools

import jax
import jax.numpy as jnp
from jax import lax
from jax.experimental import pallas as pl
from jax.experimental.pallas import tpu as pltpu
from jax.experimental.pallas import tpu_sc as plsc

B = 4096
K = 200
D = 128
L = 16
NCHUNK = D // L


_BBA = 128
_ESMALL = 1024


def _stage_a_body(hidx_ref, ridx_ref, pidx_ref, e_ref, rel_ref,
                  t_ref, tp_ref):
    i = pl.program_id(0)

    def per_8(j8, _):
        j0 = j8 * 8
        b0 = i * _BBA + j0
        ts, tps = [], []
        for u in range(8):
            b = b0 + u
            h = e_ref[pl.ds(hidx_ref[b], 1), :]
            R = rel_ref[pl.ds(ridx_ref[b], 1)]
            t = lax.dot_general(h, R, (((1,), (1,)), ((0,), (0,))),
                                preferred_element_type=jnp.float32)
            p = e_ref[pl.ds(pidx_ref[b], 1), :]
            ts.append(t)
            tps.append(t * p)
        t_ref[pl.ds(j0, 8), :] = jnp.concatenate(ts, 0)
        tp_ref[pl.ds(j0, 8), :] = jnp.concatenate(tps, 0)
        return ()

    lax.fori_loop(0, _BBA // 8, per_8, ())


def _stage_a(hidx, ridx, pidx, e_small, relation_embedding):
    nb = hidx.shape[0]
    grid_spec = pltpu.PrefetchScalarGridSpec(
        num_scalar_prefetch=3,
        grid=(nb // _BBA,),
        in_specs=[
            pl.BlockSpec((_ESMALL, D), lambda i, h, r, p: (0, 0)),
            pl.BlockSpec((1000, D, D), lambda i, h, r, p: (0, 0, 0)),
        ],
        out_specs=[
            pl.BlockSpec((_BBA, D), lambda i, h, r, p: (i, 0)),
            pl.BlockSpec((_BBA, D), lambda i, h, r, p: (i, 0)),
        ],
    )
    fn = pl.pallas_call(
        _stage_a_body,
        grid_spec=grid_spec,
        out_shape=[
            jax.ShapeDtypeStruct((nb, D), jnp.float32),
            jax.ShapeDtypeStruct((nb, D), jnp.float32),
        ],
        compiler_params=pltpu.CompilerParams(
            vmem_limit_bytes=110 * 1024 * 1024),
    )
    return fn(hidx, ridx, pidx, e_small, relation_embedding)


@functools.cache
def _make_stage_b(nb):
    info = plsc.get_sparse_core_info()
    nw = info.num_cores * info.num_subcores
    b_per_w = nb // nw
    mesh = plsc.VectorSubcoreMesh(core_axis_name="c", subcore_axis_name="s")

    KP = 208
    NG = KP // L

    @functools.partial(
        pl.kernel,
        mesh=mesh,
        out_type=jax.ShapeDtypeStruct((nb * K,), jnp.float32),
        scratch_types=[
            pltpu.VMEM((K,), jnp.int32),
            pltpu.VMEM((K,), jnp.int32),
            pltpu.VMEM((KP, D), jnp.float32),
            pltpu.VMEM((KP, D), jnp.float32),
            pltpu.VMEM((D,), jnp.float32),
            pltpu.VMEM((D,), jnp.float32),
            pltpu.VMEM((KP,), jnp.float32),
            pltpu.VMEM((KP,), jnp.float32),
            pltpu.SemaphoreType.DMA,
            pltpu.SemaphoreType.DMA,
            pltpu.SemaphoreType.DMA,
            pltpu.SemaphoreType.DMA,
            pltpu.SemaphoreType.DMA,
            pltpu.SemaphoreType.DMA,
        ],
    )
    def stage_b(table_hbm, idx_hbm, t_hbm, out_hbm,
                idx_v0, idx_v1, rows_v0, rows_v1, t_v0, t_v1,
                scores_v0, scores_v1,
                sa0, sa1, sb0, sb1, so0, so1):
        idx_v = [idx_v0, idx_v1]
        rows_v = [rows_v0, rows_v1]
        t_v = [t_v0, t_v1]
        scores_v = [scores_v0, scores_v1]
        sa = [sa0, sa1]
        sb = [sb0, sb1]
        so = [so0, so1]
        wid = lax.axis_index("s") * info.num_cores + lax.axis_index("c")
        base = wid * b_per_w
        lane = lax.iota(jnp.int32, L)
        lane_masks = [lane == j for j in range(L)]
        perm_idx = [(lane ^ s).reshape(L, 1) for s in (8, 4, 2, 1)]
        dnums = lax.GatherDimensionNumbers(
            offset_dims=(), collapsed_slice_dims=(0,), start_index_map=(0,))

        def lane_sum(v):
            for idx in perm_idx:
                v = v + lax.gather(
                    v, idx, dnums, (1,),
                    mode=lax.GatherScatterMode.PROMISE_IN_BOUNDS)
            return v

        def stage_copies(i, s):
            b = base + i
            bK = pl.multiple_of(b * K, 8)
            bD = pl.multiple_of(b * D, 8)
            return (
                pltpu.make_async_copy(idx_hbm.at[pl.ds(bK, K)],
                                      idx_v[s], sa[s]),
                pltpu.make_async_copy(t_hbm.at[pl.ds(bD, D)],
                                      t_v[s], sa[s]),
            )

        def row_gather(s):
            return pltpu.make_async_copy(
                table_hbm.at[idx_v[s]],
                rows_v[s].at[pl.ds(0, K)], sb[s])

        def out_copy(i, s):
            b = base + i
            bK = pl.multiple_of(b * K, 8)
            return pltpu.make_async_copy(
                scores_v[s].at[pl.ds(0, K)], out_hbm.at[pl.ds(bK, K)], so[s])

        c0, c1 = stage_copies(0, 0)
        c0.start(); c1.start(); c0.wait(); c1.wait()
        row_gather(0).start()
        c0, c1 = stage_copies(1, 1)
        c0.start(); c1.start()

        def do_one(i, s):
            @pl.when(i + 1 < b_per_w)
            def _():
                c0, c1 = stage_copies(i + 1, 1 - s)
                c0.wait(); c1.wait()
                row_gather(1 - s).start()

            row_gather(s).wait()
            t_chunks = [t_v[s][pl.ds(c * L, L)] for c in range(NCHUNK)]

            @pl.when(i + 2 < b_per_w)
            def _():
                c0, c1 = stage_copies(i + 2, s)
                c0.start(); c1.start()

            @pl.when(i >= 2)
            def _():
                out_copy(i - 2, s).wait()

            def per_group(g, _):
                k0 = pl.multiple_of(g * L, L)
                vec = jnp.zeros((L,), jnp.float32)
                for j in range(L):
                    k = k0 + j
                    acc = rows_v[s][k, pl.ds(0, L)] * t_chunks[0]
                    for c in range(1, NCHUNK):
                        acc = acc + rows_v[s][k, pl.ds(c * L, L)] * t_chunks[c]
                    sc = lane_sum(acc)
                    vec = jnp.where(lane_masks[j], sc, vec)
                scores_v[s][pl.ds(k0, L)] = vec
                return ()

            lax.fori_loop(0, NG, per_group, (), unroll=2)
            out_copy(i, s).start()

        def per_pair(g, _):
            do_one(g * 2, 0)
            do_one(g * 2 + 1, 1)
            return ()

        lax.fori_loop(0, b_per_w // 2, per_pair, ())
        out_copy(b_per_w - 2, 0).wait()
        out_copy(b_per_w - 1, 1).wait()

    return stage_b


def _log_sigmoid(x):
    return jnp.minimum(x, 0.0) - jnp.log1p(jnp.exp(-jnp.abs(x)))


def _stage_c_body(nega_ref, tp_ref, posi1_ref, nega1_ref):
    p = jnp.sum(tp_ref[...], axis=1)
    posi1_ref[...] = _log_sigmoid(p)
    n = nega_ref[...]
    nega1_ref[...] = jnp.mean(_log_sigmoid(-n), axis=1)


def _stage_c(nega_score, tp):
    nb = tp.shape[0]
    BB = 256
    return pl.pallas_call(
        _stage_c_body,
        grid=(nb // BB,),
        in_specs=[
            pl.BlockSpec((BB, K), lambda i: (i, 0)),
            pl.BlockSpec((BB, D), lambda i: (i, 0)),
        ],
        out_specs=[
            pl.BlockSpec((BB,), lambda i: (i,)),
            pl.BlockSpec((BB,), lambda i: (i,)),
        ],
        out_shape=[
            jax.ShapeDtypeStruct((nb,), jnp.float32),
            jax.ShapeDtypeStruct((nb,), jnp.float32),
        ],
    )(nega_score, tp)


_CHUNKS = (1536, 2560)


def kernel(head_part, tail_part, entity_embedding, relation_embedding):
    hidx = head_part[:, 0].astype(jnp.int32)
    ridx = head_part[:, 1].astype(jnp.int32)
    pidx = head_part[:, 2].astype(jnp.int32)
    tail_idx = tail_part.astype(jnp.int32).reshape(-1)
    e_small = entity_embedding[:_ESMALL]

    posi1, nega1, nega = [], [], []
    lo = 0
    for nb in _CHUNKS:
        t, tp = _stage_a(hidx[lo:lo + nb], ridx[lo:lo + nb],
                         pidx[lo:lo + nb], e_small, relation_embedding)
        ns = _make_stage_b(nb)(entity_embedding,
                               tail_idx[lo * K:(lo + nb) * K],
                               t.reshape(-1)).reshape(nb, K)
        p1, n1 = _stage_c(ns, tp)
        posi1.append(p1)
        nega1.append(n1)
        nega.append(ns)
        lo += nb
    return (jnp.concatenate(posi1), jnp.concatenate(nega1),
            jnp.concatenate(nega, axis=0))


# speedup vs baseline: 1.0316x; 1.0316x over previous
import functools

import jax
import jax.numpy as jnp
from jax import lax
from jax.experimental import pallas as pl
from jax.experimental.pallas import tpu as pltpu
from jax.experimental.pallas import tpu_sc as plsc

B = 4096
K = 200
D = 128
L = 16
NCHUNK = D // L


_BBA = 128
_ESMALL = 1024


def _stage_a_body(hidx_ref, ridx_ref, pidx_ref, e_ref, rel_ref,
                  t_ref, tp_ref):
    i = pl.program_id(0)

    def per_8(j8, _):
        j0 = j8 * 8
        b0 = i * _BBA + j0
        ts, tps = [], []
        for u in range(8):
            b = b0 + u
            h = e_ref[pl.ds(hidx_ref[b], 1), :]
            R = rel_ref[pl.ds(ridx_ref[b], 1)]
            t = lax.dot_general(h, R, (((1,), (1,)), ((0,), (0,))),
                                preferred_element_type=jnp.float32)
            p = e_ref[pl.ds(pidx_ref[b], 1), :]
            ts.append(t)
            tps.append(t * p)
        t_ref[pl.ds(j0, 8), :] = jnp.concatenate(ts, 0)
        tp_ref[pl.ds(j0, 8), :] = jnp.concatenate(tps, 0)
        return ()

    lax.fori_loop(0, _BBA // 8, per_8, ())


def _stage_a(hidx, ridx, pidx, e_small, relation_embedding):
    nb = hidx.shape[0]
    grid_spec = pltpu.PrefetchScalarGridSpec(
        num_scalar_prefetch=3,
        grid=(nb // _BBA,),
        in_specs=[
            pl.BlockSpec((_ESMALL, D), lambda i, h, r, p: (0, 0)),
            pl.BlockSpec((1000, D, D), lambda i, h, r, p: (0, 0, 0)),
        ],
        out_specs=[
            pl.BlockSpec((_BBA, D), lambda i, h, r, p: (i, 0)),
            pl.BlockSpec((_BBA, D), lambda i, h, r, p: (i, 0)),
        ],
    )
    fn = pl.pallas_call(
        _stage_a_body,
        grid_spec=grid_spec,
        out_shape=[
            jax.ShapeDtypeStruct((nb, D), jnp.float32),
            jax.ShapeDtypeStruct((nb, D), jnp.float32),
        ],
        compiler_params=pltpu.CompilerParams(
            vmem_limit_bytes=110 * 1024 * 1024),
    )
    return fn(hidx, ridx, pidx, e_small, relation_embedding)


@functools.cache
def _make_stage_b(nb):
    info = plsc.get_sparse_core_info()
    nw = info.num_cores * info.num_subcores
    b_per_w = nb // nw
    mesh = plsc.VectorSubcoreMesh(core_axis_name="c", subcore_axis_name="s")

    KP = 208
    NG = KP // L

    @functools.partial(
        pl.kernel,
        mesh=mesh,
        out_type=jax.ShapeDtypeStruct((nb * K,), jnp.float32),
        scratch_types=[
            pltpu.VMEM((K,), jnp.int32),
            pltpu.VMEM((K,), jnp.int32),
            pltpu.VMEM((KP, D), jnp.float32),
            pltpu.VMEM((KP, D), jnp.float32),
            pltpu.VMEM((D,), jnp.float32),
            pltpu.VMEM((D,), jnp.float32),
            pltpu.VMEM((KP,), jnp.float32),
            pltpu.VMEM((KP,), jnp.float32),
            pltpu.SemaphoreType.DMA,
            pltpu.SemaphoreType.DMA,
            pltpu.SemaphoreType.DMA,
            pltpu.SemaphoreType.DMA,
            pltpu.SemaphoreType.DMA,
            pltpu.SemaphoreType.DMA,
        ],
    )
    def stage_b(table_hbm, idx_hbm, t_hbm, out_hbm,
                idx_v0, idx_v1, rows_v0, rows_v1, t_v0, t_v1,
                scores_v0, scores_v1,
                sa0, sa1, sb0, sb1, so0, so1):
        idx_v = [idx_v0, idx_v1]
        rows_v = [rows_v0, rows_v1]
        t_v = [t_v0, t_v1]
        scores_v = [scores_v0, scores_v1]
        sa = [sa0, sa1]
        sb = [sb0, sb1]
        so = [so0, so1]
        wid = lax.axis_index("s") * info.num_cores + lax.axis_index("c")
        base = wid * b_per_w
        lane = lax.iota(jnp.int32, L)
        lane_masks = [lane == j for j in range(L)]
        perm_idx = [(lane ^ s).reshape(L, 1) for s in (8, 4, 2, 1)]
        dnums = lax.GatherDimensionNumbers(
            offset_dims=(), collapsed_slice_dims=(0,), start_index_map=(0,))

        def lane_sum(v):
            for idx in perm_idx:
                v = v + lax.gather(
                    v, idx, dnums, (1,),
                    mode=lax.GatherScatterMode.PROMISE_IN_BOUNDS)
            return v

        def stage_copies(i, s):
            b = base + i
            bK = pl.multiple_of(b * K, 8)
            bD = pl.multiple_of(b * D, 8)
            return (
                pltpu.make_async_copy(idx_hbm.at[pl.ds(bK, K)],
                                      idx_v[s], sa[s]),
                pltpu.make_async_copy(t_hbm.at[pl.ds(bD, D)],
                                      t_v[s], sa[s]),
            )

        def row_gather(s):
            return pltpu.make_async_copy(
                table_hbm.at[idx_v[s]],
                rows_v[s].at[pl.ds(0, K)], sb[s])

        def out_copy(i, s):
            b = base + i
            bK = pl.multiple_of(b * K, 8)
            return pltpu.make_async_copy(
                scores_v[s].at[pl.ds(0, K)], out_hbm.at[pl.ds(bK, K)], so[s])

        c0, c1 = stage_copies(0, 0)
        c0.start(); c1.start(); c0.wait(); c1.wait()
        row_gather(0).start()
        c0, c1 = stage_copies(1, 1)
        c0.start(); c1.start()

        def do_one(i, s):
            @pl.when(i + 1 < b_per_w)
            def _():
                c0, c1 = stage_copies(i + 1, 1 - s)
                c0.wait(); c1.wait()
                row_gather(1 - s).start()

            row_gather(s).wait()
            t_chunks = [t_v[s][pl.ds(c * L, L)] for c in range(NCHUNK)]

            @pl.when(i + 2 < b_per_w)
            def _():
                c0, c1 = stage_copies(i + 2, s)
                c0.start(); c1.start()

            @pl.when(i >= 2)
            def _():
                out_copy(i - 2, s).wait()

            def per_group(g, _):
                k0 = pl.multiple_of(g * L, L)
                vec = jnp.zeros((L,), jnp.float32)
                for j in range(L):
                    k = k0 + j
                    acc = rows_v[s][k, pl.ds(0, L)] * t_chunks[0]
                    for c in range(1, NCHUNK):
                        acc = acc + rows_v[s][k, pl.ds(c * L, L)] * t_chunks[c]
                    sc = lane_sum(acc)
                    vec = jnp.where(lane_masks[j], sc, vec)
                scores_v[s][pl.ds(k0, L)] = vec
                return ()

            lax.fori_loop(0, NG, per_group, (), unroll=2)
            out_copy(i, s).start()

        def per_pair(g, _):
            do_one(g * 2, 0)
            do_one(g * 2 + 1, 1)
            return ()

        lax.fori_loop(0, b_per_w // 2, per_pair, ())
        out_copy(b_per_w - 2, 0).wait()
        out_copy(b_per_w - 1, 1).wait()

    return stage_b


def _log_sigmoid(x):
    return jnp.minimum(x, 0.0) - jnp.log1p(jnp.exp(-jnp.abs(x)))


def _stage_c_body(nega_ref, tp_ref, posi1_ref, nega1_ref):
    p = jnp.sum(tp_ref[...], axis=1)
    posi1_ref[...] = _log_sigmoid(p)
    n = nega_ref[...]
    nega1_ref[...] = jnp.mean(_log_sigmoid(-n), axis=1)


def _stage_c(nega_score, tp):
    nb = tp.shape[0]
    BB = 256
    return pl.pallas_call(
        _stage_c_body,
        grid=(nb // BB,),
        in_specs=[
            pl.BlockSpec((BB, K), lambda i: (i, 0)),
            pl.BlockSpec((BB, D), lambda i: (i, 0)),
        ],
        out_specs=[
            pl.BlockSpec((BB,), lambda i: (i,)),
            pl.BlockSpec((BB,), lambda i: (i,)),
        ],
        out_shape=[
            jax.ShapeDtypeStruct((nb,), jnp.float32),
            jax.ShapeDtypeStruct((nb,), jnp.float32),
        ],
    )(nega_score, tp)


_CHUNKS = (2048, 2048)


def kernel(head_part, tail_part, entity_embedding, relation_embedding):
    hidx = head_part[:, 0].astype(jnp.int32)
    ridx = head_part[:, 1].astype(jnp.int32)
    pidx = head_part[:, 2].astype(jnp.int32)
    tail_idx = tail_part.astype(jnp.int32).reshape(-1)
    e_small = entity_embedding[:_ESMALL]

    posi1, nega1, nega = [], [], []
    lo = 0
    for nb in _CHUNKS:
        t, tp = _stage_a(hidx[lo:lo + nb], ridx[lo:lo + nb],
                         pidx[lo:lo + nb], e_small, relation_embedding)
        ns = _make_stage_b(nb)(entity_embedding,
                               tail_idx[lo * K:(lo + nb) * K],
                               t.reshape(-1)).reshape(nb, K)
        p1, n1 = _stage_c(ns, tp)
        posi1.append(p1)
        nega1.append(n1)
        nega.append(ns)
        lo += nb
    return (jnp.concatenate(posi1), jnp.concatenate(nega1),
            jnp.concatenate(nega, axis=0))
